# baseline (speedup 1.0000x reference)
import jax
import jax.numpy as jnp
from jax import lax
from jax.experimental import pallas as pl
from jax.experimental.pallas import tpu as pltpu
from jax.experimental.pallas import tpu_sc as plsc

BATCH = 16384
NF = 26
D = 16
VOCAB = 100000
XW = 2 * NF
OUT_W = NF * D + NF
BLK = 128
VPB = BLK // D
NBLK = VOCAB // VPB

NC = 2
NS = 16
NW = NC * NS
B_PER_W = BATCH // NW
BC = 128
N_CHUNK = B_PER_W // BC
L = 16


def _body(x_hbm, tab_hbm, out_hbm, xv, blk_v, low_v, bbuf, stage_v, gsem, osem):
    wid = lax.axis_index("s") * NC + lax.axis_index("c")
    iota = lax.iota(jnp.int32, L)

    def chunk(c, carry):
        base = wid * B_PER_W + c * BC

        pltpu.sync_copy(x_hbm.at[pl.ds(base, BC), :], xv)

        for j in range(NF * BC // L):
            f = j // (BC // L)
            b0 = (j % (BC // L)) * L
            vals = plsc.load_gather(
                xv, [b0 + iota, jnp.full((L,), f, jnp.int32)]
            )
            blk_v[f, pl.ds(b0, L)] = vals >> 3
            low_v[f, pl.ds(b0, L)] = (vals & 7) << 4

        for j in range(NF * BC // L):
            col = j // (BC // L)
            b0 = (j % (BC // L)) * L
            vals = plsc.load_gather(
                xv, [b0 + iota, jnp.full((L,), NF + col, jnp.int32)]
            )
            stage_v[NF * D + col, pl.ds(b0, L)] = vals.astype(jnp.float32)

        pltpu.async_copy(
            tab_hbm.at[0].at[blk_v.at[0]], bbuf.at[pl.ds(0, BC), :], gsem
        )

        def feat(f, carry2):
            nxt = f + 1

            @pl.when(nxt < NF)
            def _():
                pltpu.async_copy(
                    tab_hbm.at[nxt].at[blk_v.at[nxt]],
                    bbuf.at[pl.ds((nxt % 2) * BC, BC), :],
                    gsem,
                )

            pltpu.make_async_copy(
                tab_hbm.at[0].at[blk_v.at[0]],
                bbuf.at[pl.ds((f % 2) * BC, BC), :],
                gsem,
            ).wait()

            bufbase = (f % 2) * BC
            for b0 in range(BC // L):
                rows = bufbase + b0 * L + iota
                lowvec = low_v[f, pl.ds(b0 * L, L)]
                for d in range(D):
                    vals = plsc.load_gather(bbuf, [rows, lowvec + d])
                    stage_v[f * D + d, pl.ds(b0 * L, L)] = vals
            return carry2

        lax.fori_loop(0, NF, feat, 0)

        pltpu.async_copy(stage_v, out_hbm.at[:, pl.ds(base, BC)], osem).wait()
        return carry

    lax.fori_loop(0, N_CHUNK, chunk, 0)


@jax.jit
def _emb_lookup(x, tab5):
    run = pl.kernel(
        _body,
        out_type=jax.ShapeDtypeStruct((OUT_W, BATCH), jnp.float32),
        mesh=plsc.VectorSubcoreMesh(
            core_axis_name="c", subcore_axis_name="s", num_cores=NC,
            num_subcores=NS,
        ),
        scratch_types=[
            pltpu.VMEM((BC, XW), jnp.int32),
            pltpu.VMEM((NF, BC), jnp.int32),
            pltpu.VMEM((NF, BC), jnp.int32),
            pltpu.VMEM((2 * BC, BLK), jnp.float32),
            pltpu.VMEM((OUT_W, BC), jnp.float32),
            pltpu.SemaphoreType.DMA,
            pltpu.SemaphoreType.DMA,
        ],
        compiler_params=pltpu.CompilerParams(
            use_tc_tiling_on_sc=True, needs_layout_passes=False
        ),
    )
    return run(x, tab5)


def kernel(x, tables):
    return _emb_lookup(x, tables.reshape(NF, NBLK, BLK)).T


# speedup vs baseline: 1.0618x; 1.0618x over previous
import jax
import jax.numpy as jnp
from jax import lax
from jax.experimental import pallas as pl
from jax.experimental.pallas import tpu as pltpu
from jax.experimental.pallas import tpu_sc as plsc

BATCH = 16384
NF = 26
D = 16
VOCAB = 100000
XW = 2 * NF
OUT_W = NF * D + NF
BLK = 128
VPB = BLK // D
NBLK = 12504
WIN = 128
NWIN_FULL = VOCAB // WIN
N_ITEMS = NF * NWIN_FULL
TAIL_V0 = NWIN_FULL * WIN
TAIL_N = VOCAB - TAIL_V0

NC = 2
NS = 16
NW = NC * NS
B_PER_W = BATCH // NW
BC = 128
N_CHUNK = B_PER_W // BC
L = 16


def _tr_body(tab_t_hbm, tail_hbm, out_hbm, in_v, out_v, gsem, osem):
    wid = lax.axis_index("s") * NC + lax.axis_index("c")
    iota = lax.iota(jnp.int32, L)
    ni = (N_ITEMS - wid + NW - 1) // NW

    def load(item, slot):
        f = item // NWIN_FULL
        w0 = item - f * NWIN_FULL
        return pltpu.async_copy(
            tab_t_hbm.at[f, :, pl.ds(w0 * WIN, WIN)],
            in_v.at[pl.ds(slot * D, D), :],
            gsem,
        )

    load(wid, 0)

    def step(i, carry):
        item = wid + i * NW

        @pl.when(i + 1 < ni)
        def _():
            load(item + NW, (i + 1) % 2)

        pltpu.make_async_copy(
            tab_t_hbm.at[0, :, pl.ds(0, WIN)],
            in_v.at[pl.ds(0, D), :],
            gsem,
        ).wait()

        s = (i % 2) * D
        for r in range(D):
            for j0 in range(VPB):
                vals = plsc.load_gather(
                    in_v, [s + iota, jnp.full((L,), VPB * r + j0, jnp.int32)]
                )
                out_v[s + r, pl.ds(j0 * L, L)] = vals

        @pl.when(i >= 2)
        def _():
            pltpu.make_async_copy(
                tab_t_hbm.at[0, :, pl.ds(0, WIN)],
                out_v.at[pl.ds(0, D), :],
                osem,
            ).wait()

        f = item // NWIN_FULL
        w0 = item - f * NWIN_FULL
        pltpu.async_copy(
            out_v.at[pl.ds(s, D), :],
            out_hbm.at[f, pl.ds(w0 * D, D), :],
            osem,
        )
        return carry

    lax.fori_loop(0, ni, step, 0)

    for _ in range(2):
        pltpu.make_async_copy(
            tab_t_hbm.at[0, :, pl.ds(0, WIN)],
            out_v.at[pl.ds(0, D), :],
            osem,
        ).wait()

    @pl.when(wid < NF)
    def _():
        pltpu.sync_copy(tail_hbm.at[wid], in_v.at[pl.ds(0, VPB), :])
        pltpu.sync_copy(
            in_v.at[pl.ds(0, VPB), :],
            out_hbm.at[wid, pl.ds(NWIN_FULL * D, VPB), :],
        )


def _emb_body(x_hbm, tab_hbm, out_hbm, xv, blk_v, low_v, bbuf, stage_v, gsem, osem):
    wid = lax.axis_index("s") * NC + lax.axis_index("c")
    iota = lax.iota(jnp.int32, L)

    def chunk(c, carry):
        base = wid * B_PER_W + c * BC

        pltpu.sync_copy(x_hbm.at[pl.ds(base, BC), :], xv)

        for j in range(NF * BC // L):
            f = j // (BC // L)
            b0 = (j % (BC // L)) * L
            vals = plsc.load_gather(
                xv, [b0 + iota, jnp.full((L,), f, jnp.int32)]
            )
            blk_v[f, pl.ds(b0, L)] = vals >> 3
            low_v[f, pl.ds(b0, L)] = (vals & 7) << 4

        for j in range(NF * BC // L):
            col = j // (BC // L)
            b0 = (j % (BC // L)) * L
            vals = plsc.load_gather(
                xv, [b0 + iota, jnp.full((L,), NF + col, jnp.int32)]
            )
            stage_v[NF * D + col, pl.ds(b0, L)] = vals.astype(jnp.float32)

        pltpu.async_copy(
            tab_hbm.at[0].at[blk_v.at[0]], bbuf.at[pl.ds(0, BC), :], gsem
        )

        def feat(f, carry2):
            nxt = f + 1

            @pl.when(nxt < NF)
            def _():
                pltpu.async_copy(
                    tab_hbm.at[nxt].at[blk_v.at[nxt]],
                    bbuf.at[pl.ds((nxt % 2) * BC, BC), :],
                    gsem,
                )

            pltpu.make_async_copy(
                tab_hbm.at[0].at[blk_v.at[0]],
                bbuf.at[pl.ds((f % 2) * BC, BC), :],
                gsem,
            ).wait()

            bufbase = (f % 2) * BC
            for b0 in range(BC // L):
                rows = bufbase + b0 * L + iota
                lowvec = low_v[f, pl.ds(b0 * L, L)]
                for d in range(D):
                    vals = plsc.load_gather(bbuf, [rows, lowvec + d])
                    stage_v[f * D + d, pl.ds(b0 * L, L)] = vals
            return carry2

        lax.fori_loop(0, NF, feat, 0)

        pltpu.async_copy(stage_v, out_hbm.at[:, pl.ds(base, BC)], osem).wait()
        return carry

    lax.fori_loop(0, N_CHUNK, chunk, 0)


_MESH = plsc.VectorSubcoreMesh(
    core_axis_name="c", subcore_axis_name="s", num_cores=NC, num_subcores=NS
)
_PARAMS = pltpu.CompilerParams(
    use_tc_tiling_on_sc=True, needs_layout_passes=False
)


@jax.jit
def _transpose_tables(tab_t, tail8):
    run = pl.kernel(
        _tr_body,
        out_type=jax.ShapeDtypeStruct((NF, NBLK, BLK), jnp.float32),
        mesh=_MESH,
        scratch_types=[
            pltpu.VMEM((2 * D, WIN), jnp.float32),
            pltpu.VMEM((2 * D, WIN), jnp.float32),
            pltpu.SemaphoreType.DMA,
            pltpu.SemaphoreType.DMA,
        ],
        compiler_params=_PARAMS,
    )
    return run(tab_t, tail8)


@jax.jit
def _emb_lookup(x, tab5):
    run = pl.kernel(
        _emb_body,
        out_type=jax.ShapeDtypeStruct((OUT_W, BATCH), jnp.float32),
        mesh=_MESH,
        scratch_types=[
            pltpu.VMEM((BC, XW), jnp.int32),
            pltpu.VMEM((NF, BC), jnp.int32),
            pltpu.VMEM((NF, BC), jnp.int32),
            pltpu.VMEM((2 * BC, BLK), jnp.float32),
            pltpu.VMEM((OUT_W, BC), jnp.float32),
            pltpu.SemaphoreType.DMA,
            pltpu.SemaphoreType.DMA,
        ],
        compiler_params=_PARAMS,
    )
    return run(x, tab5)


def kernel(x, tables):
    tail8 = jnp.pad(
        tables[:, TAIL_V0:, :].reshape(NF, TAIL_N * D // BLK, BLK),
        ((0, 0), (0, VPB - TAIL_N * D // BLK), (0, 0)),
    )
    tab_lin = _transpose_tables(tables.transpose(0, 2, 1), tail8)
    return _emb_lookup(x, tab_lin).T


# speedup vs baseline: 1.0660x; 1.0040x over previous
import jax
import jax.numpy as jnp
from jax import lax
from jax.experimental import pallas as pl
from jax.experimental.pallas import tpu as pltpu
from jax.experimental.pallas import tpu_sc as plsc

BATCH = 16384
NF = 26
D = 16
VOCAB = 100000
XW = 2 * NF
OUT_W = NF * D + NF
BLK = 128
VPB = BLK // D
NBLK = 12512
WIN = 1024
NARROW = 640
NCH = 98
N_ITEMS = NF * NCH
TAIL_V0 = 97 * WIN + NARROW
TAIL_N = VOCAB - TAIL_V0

NC = 2
NS = 16
NW = NC * NS
B_PER_W = BATCH // NW
BC = 128
N_CHUNK = B_PER_W // BC
L = 16


def _tr_body(tab_t_hbm, tail_hbm, out_hbm, in_v, out_v, gsem, osem):
    wid = lax.axis_index("s") * NC + lax.axis_index("c")
    iota = lax.iota(jnp.int32, L)
    ni = (N_ITEMS - wid + NW - 1) // NW

    def fw(item):
        f = item // NCH
        k = item - f * NCH
        return f, k

    def load(item, slot):
        f, k = fw(item)

        @pl.when(k < NCH - 1)
        def _():
            for g in range(2):
                pltpu.async_copy(
                    tab_t_hbm.at[f, pl.ds(g * VPB, VPB), pl.ds(k * WIN, WIN)],
                    in_v.at[pl.ds(slot * D + g * VPB, VPB), :],
                    gsem,
                )

        @pl.when(k == NCH - 1)
        def _():
            for g in range(2):
                pltpu.async_copy(
                    tab_t_hbm.at[
                        f, pl.ds(g * VPB, VPB), pl.ds(97 * WIN, NARROW)
                    ],
                    in_v.at[pl.ds(slot * D + g * VPB, VPB), pl.ds(0, NARROW)],
                    gsem,
                )

    def absorb_in(item):
        _, k = fw(item)

        @pl.when(k < NCH - 1)
        def _():
            for g in range(2):
                pltpu.make_async_copy(
                    tab_t_hbm.at[0, pl.ds(0, VPB), pl.ds(0, WIN)],
                    in_v.at[pl.ds(g * VPB, VPB), :],
                    gsem,
                ).wait()

        @pl.when(k == NCH - 1)
        def _():
            for g in range(2):
                pltpu.make_async_copy(
                    tab_t_hbm.at[0, pl.ds(0, VPB), pl.ds(0, NARROW)],
                    in_v.at[pl.ds(g * VPB, VPB), pl.ds(0, NARROW)],
                    gsem,
                ).wait()

    def absorb_out(item):
        _, k = fw(item)

        @pl.when(k < NCH - 1)
        def _():
            pltpu.make_async_copy(
                out_hbm.at[0, pl.ds(0, WIN // VPB), :],
                out_v.at[pl.ds(0, WIN // VPB), :],
                osem,
            ).wait()

        @pl.when(k == NCH - 1)
        def _():
            pltpu.make_async_copy(
                out_hbm.at[0, pl.ds(0, NARROW // VPB), :],
                out_v.at[pl.ds(0, NARROW // VPB), :],
                osem,
            ).wait()

    @pl.when(ni > 0)
    def _():
        load(wid, 0)

    @pl.when(ni > 1)
    def _():
        load(wid + NW, 1)

    def step(i, carry):
        item = wid + i * NW
        f, k = fw(item)
        nrows = lax.select(
            k < NCH - 1,
            jnp.int32(WIN // VPB),
            jnp.int32(NARROW // VPB),
        )

        @pl.when(i + 2 < ni)
        def _():
            load(item + 2 * NW, (i + 2) % 3)

        absorb_in(item)

        s_in = (i % 3) * D
        s_out = (i % 2) * (WIN // VPB)

        def row(r, carry2):
            for q in range(VPB):
                vals = plsc.load_gather(
                    in_v,
                    [s_in + iota, jnp.full((L,), VPB, jnp.int32) * r + q],
                )
                out_v[s_out + r, pl.ds(q * L, L)] = vals
            return carry2

        lax.fori_loop(0, nrows, row, 0)

        @pl.when(i >= 2)
        def _():
            absorb_out(item - 2 * NW)

        @pl.when(k < NCH - 1)
        def _():
            pltpu.async_copy(
                out_v.at[pl.ds(s_out, WIN // VPB), :],
                out_hbm.at[f, pl.ds(k * (WIN // VPB), WIN // VPB), :],
                osem,
            )

        @pl.when(k == NCH - 1)
        def _():
            pltpu.async_copy(
                out_v.at[pl.ds(s_out, NARROW // VPB), :],
                out_hbm.at[f, pl.ds(97 * (WIN // VPB), NARROW // VPB), :],
                osem,
            )
        return carry

    lax.fori_loop(0, ni, step, 0)

    @pl.when(ni >= 2)
    def _():
        absorb_out(wid + (ni - 2) * NW)

    @pl.when(ni >= 1)
    def _():
        absorb_out(wid + (ni - 1) * NW)

    @pl.when(wid < NF)
    def _():
        pltpu.sync_copy(
            tail_hbm.at[wid], in_v.at[pl.ds(0, VPB), pl.ds(0, BLK)]
        )
        pltpu.sync_copy(
            in_v.at[pl.ds(0, VPB), pl.ds(0, BLK)],
            out_hbm.at[wid, pl.ds(TAIL_V0 // VPB, VPB), :],
        )


def _emb_body(x_hbm, tab_hbm, out_hbm, xv, blk_v, low_v, bbuf, stage_v, gsem, osem):
    wid = lax.axis_index("s") * NC + lax.axis_index("c")
    iota = lax.iota(jnp.int32, L)

    def chunk(c, carry):
        base = wid * B_PER_W + c * BC

        pltpu.sync_copy(x_hbm.at[pl.ds(base, BC), :], xv)

        for j in range(NF * BC // L):
            f = j // (BC // L)
            b0 = (j % (BC // L)) * L
            vals = plsc.load_gather(
                xv, [b0 + iota, jnp.full((L,), f, jnp.int32)]
            )
            blk_v[f, pl.ds(b0, L)] = vals >> 3
            low_v[f, pl.ds(b0, L)] = (vals & 7) << 4

        for j in range(NF * BC // L):
            col = j // (BC // L)
            b0 = (j % (BC // L)) * L
            vals = plsc.load_gather(
                xv, [b0 + iota, jnp.full((L,), NF + col, jnp.int32)]
            )
            stage_v[NF * D + col, pl.ds(b0, L)] = vals.astype(jnp.float32)

        pltpu.async_copy(
            tab_hbm.at[0].at[blk_v.at[0]], bbuf.at[pl.ds(0, BC), :], gsem
        )

        def feat(f, carry2):
            nxt = f + 1

            @pl.when(nxt < NF)
            def _():
                pltpu.async_copy(
                    tab_hbm.at[nxt].at[blk_v.at[nxt]],
                    bbuf.at[pl.ds((nxt % 2) * BC, BC), :],
                    gsem,
                )

            pltpu.make_async_copy(
                tab_hbm.at[0].at[blk_v.at[0]],
                bbuf.at[pl.ds((f % 2) * BC, BC), :],
                gsem,
            ).wait()

            bufbase = (f % 2) * BC
            for b0 in range(BC // L):
                rows = bufbase + b0 * L + iota
                lowvec = low_v[f, pl.ds(b0 * L, L)]
                for d in range(D):
                    vals = plsc.load_gather(bbuf, [rows, lowvec + d])
                    stage_v[f * D + d, pl.ds(b0 * L, L)] = vals
            return carry2

        lax.fori_loop(0, NF, feat, 0)

        pltpu.async_copy(stage_v, out_hbm.at[:, pl.ds(base, BC)], osem).wait()
        return carry

    lax.fori_loop(0, N_CHUNK, chunk, 0)


_MESH = plsc.VectorSubcoreMesh(
    core_axis_name="c", subcore_axis_name="s", num_cores=NC, num_subcores=NS
)
_PARAMS = pltpu.CompilerParams(
    use_tc_tiling_on_sc=True, needs_layout_passes=False
)


@jax.jit
def _transpose_tables(tab_t, tail8):
    run = pl.kernel(
        _tr_body,
        out_type=jax.ShapeDtypeStruct((NF, NBLK, BLK), jnp.float32),
        mesh=_MESH,
        scratch_types=[
            pltpu.VMEM((3 * D, WIN), jnp.float32),
            pltpu.VMEM((2 * (WIN // VPB), BLK), jnp.float32),
            pltpu.SemaphoreType.DMA,
            pltpu.SemaphoreType.DMA,
        ],
        compiler_params=_PARAMS,
    )
    return run(tab_t, tail8)


@jax.jit
def _emb_lookup(x, tab5):
    run = pl.kernel(
        _emb_body,
        out_type=jax.ShapeDtypeStruct((OUT_W, BATCH), jnp.float32),
        mesh=_MESH,
        scratch_types=[
            pltpu.VMEM((BC, XW), jnp.int32),
            pltpu.VMEM((NF, BC), jnp.int32),
            pltpu.VMEM((NF, BC), jnp.int32),
            pltpu.VMEM((2 * BC, BLK), jnp.float32),
            pltpu.VMEM((OUT_W, BC), jnp.float32),
            pltpu.SemaphoreType.DMA,
            pltpu.SemaphoreType.DMA,
        ],
        compiler_params=_PARAMS,
    )
    return run(x, tab5)


def kernel(x, tables):
    tail8 = jnp.pad(
        tables[:, TAIL_V0:, :].reshape(NF, TAIL_N * D // BLK, BLK),
        ((0, 0), (0, VPB - TAIL_N * D // BLK), (0, 0)),
    )
    tab_lin = _transpose_tables(tables.transpose(0, 2, 1), tail8)
    return _emb_lookup(x, tab_lin).T


# speedup vs baseline: 1.0679x; 1.0018x over previous
import jax
import jax.numpy as jnp
from jax import lax
from jax.experimental import pallas as pl
from jax.experimental.pallas import tpu as pltpu
from jax.experimental.pallas import tpu_sc as plsc

BATCH = 16384
NF = 26
D = 16
VOCAB = 100000
XW = 2 * NF
OUT_W = NF * D + NF
BLK = 128
VPB = BLK // D
NBLK = 12512
WIN = 1024
NCH = 97
N_ITEMS = NF * NCH
TAIL_V0 = NCH * WIN
TAIL_ROWS = (VOCAB - TAIL_V0) * D // BLK
TAIL_PAD = 88

NC = 2
NS = 16
NW = NC * NS
B_PER_W = BATCH // NW
BC = 128
N_CHUNK = B_PER_W // BC
L = 16


def _tr_body(tab_t_hbm, tail_hbm, out_hbm, in_v, out_v, gsem, osem):
    wid = lax.axis_index("s") * NC + lax.axis_index("c")
    iota = lax.iota(jnp.int32, L)
    ni = (N_ITEMS - wid + NW - 1) // NW
    ROWS = WIN // VPB

    def load(item, slot):
        f = item // NCH
        k = item - f * NCH
        for g in range(2):
            pltpu.async_copy(
                tab_t_hbm.at[f, pl.ds(g * VPB, VPB), pl.ds(k * WIN, WIN)],
                in_v.at[pl.ds(slot * D + g * VPB, VPB), :],
                gsem,
            )

    def absorb_in():
        for g in range(2):
            pltpu.make_async_copy(
                tab_t_hbm.at[0, pl.ds(0, VPB), pl.ds(0, WIN)],
                in_v.at[pl.ds(g * VPB, VPB), :],
                gsem,
            ).wait()

    def absorb_out():
        pltpu.make_async_copy(
            out_hbm.at[0, pl.ds(0, ROWS), :],
            out_v.at[pl.ds(0, ROWS), :],
            osem,
        ).wait()

    @pl.when(ni > 0)
    def _():
        load(wid, 0)

    @pl.when(ni > 1)
    def _():
        load(wid + NW, 1)

    @pl.when(ni > 2)
    def _():
        load(wid + 2 * NW, 2)

    def step(i, carry):
        item = wid + i * NW
        f = item // NCH
        k = item - f * NCH

        @pl.when(i + 3 < ni)
        def _():
            load(item + 3 * NW, (i + 3) % 4)

        absorb_in()

        rowvec = (i % 4) * D + iota
        s_out = (i % 2) * ROWS

        def row4(rr, carry2):
            for u in range(4):
                r = rr * 4 + u
                for q in range(VPB):
                    cols = jnp.broadcast_to(VPB * r + q, (L,))
                    vals = plsc.load_gather(in_v, [rowvec, cols])
                    out_v[s_out + r, pl.ds(q * L, L)] = vals
            return carry2

        lax.fori_loop(0, ROWS // 4, row4, 0)

        @pl.when(i >= 2)
        def _():
            absorb_out()

        pltpu.async_copy(
            out_v.at[pl.ds(s_out, ROWS), :],
            out_hbm.at[f, pl.ds(k * ROWS, ROWS), :],
            osem,
        )
        return carry

    lax.fori_loop(0, ni, step, 0)

    @pl.when(ni >= 2)
    def _():
        absorb_out()

    @pl.when(ni >= 1)
    def _():
        absorb_out()

    @pl.when(wid < NF)
    def _():
        pltpu.sync_copy(tail_hbm.at[wid], out_v.at[pl.ds(0, TAIL_PAD), :])
        pltpu.sync_copy(
            out_v.at[pl.ds(0, TAIL_PAD), :],
            out_hbm.at[wid, pl.ds(TAIL_V0 // VPB, TAIL_PAD), :],
        )


def _emb_body(x_hbm, tab_hbm, out_hbm, xv, blk_v, low_v, bbuf, stage_v, gsem, osem):
    wid = lax.axis_index("s") * NC + lax.axis_index("c")
    iota = lax.iota(jnp.int32, L)

    def chunk(c, carry):
        base = wid * B_PER_W + c * BC

        pltpu.sync_copy(x_hbm.at[pl.ds(base, BC), :], xv)

        for j in range(NF * BC // L):
            f = j // (BC // L)
            b0 = (j % (BC // L)) * L
            vals = plsc.load_gather(
                xv, [b0 + iota, jnp.full((L,), f, jnp.int32)]
            )
            blk_v[f, pl.ds(b0, L)] = vals >> 3
            low_v[f, pl.ds(b0, L)] = (vals & 7) << 4

        for j in range(NF * BC // L):
            col = j // (BC // L)
            b0 = (j % (BC // L)) * L
            vals = plsc.load_gather(
                xv, [b0 + iota, jnp.full((L,), NF + col, jnp.int32)]
            )
            stage_v[NF * D + col, pl.ds(b0, L)] = vals.astype(jnp.float32)

        pltpu.async_copy(
            tab_hbm.at[0].at[blk_v.at[0]], bbuf.at[pl.ds(0, BC), :], gsem
        )

        def feat(f, carry2):
            nxt = f + 1

            @pl.when(nxt < NF)
            def _():
                pltpu.async_copy(
                    tab_hbm.at[nxt].at[blk_v.at[nxt]],
                    bbuf.at[pl.ds((nxt % 2) * BC, BC), :],
                    gsem,
                )

            pltpu.make_async_copy(
                tab_hbm.at[0].at[blk_v.at[0]],
                bbuf.at[pl.ds((f % 2) * BC, BC), :],
                gsem,
            ).wait()

            bufbase = (f % 2) * BC
            for b0 in range(BC // L):
                rows = bufbase + b0 * L + iota
                lowvec = low_v[f, pl.ds(b0 * L, L)]
                for d in range(D):
                    vals = plsc.load_gather(bbuf, [rows, lowvec + d])
                    stage_v[f * D + d, pl.ds(b0 * L, L)] = vals
            return carry2

        lax.fori_loop(0, NF, feat, 0)

        pltpu.async_copy(stage_v, out_hbm.at[:, pl.ds(base, BC)], osem).wait()
        return carry

    lax.fori_loop(0, N_CHUNK, chunk, 0)


_MESH = plsc.VectorSubcoreMesh(
    core_axis_name="c", subcore_axis_name="s", num_cores=NC, num_subcores=NS
)
_PARAMS = pltpu.CompilerParams(
    use_tc_tiling_on_sc=True, needs_layout_passes=False
)


@jax.jit
def _transpose_tables(tab_t, tail8):
    run = pl.kernel(
        _tr_body,
        out_type=jax.ShapeDtypeStruct((NF, NBLK, BLK), jnp.float32),
        mesh=_MESH,
        scratch_types=[
            pltpu.VMEM((4 * D, WIN), jnp.float32),
            pltpu.VMEM((2 * (WIN // VPB), BLK), jnp.float32),
            pltpu.SemaphoreType.DMA,
            pltpu.SemaphoreType.DMA,
        ],
        compiler_params=_PARAMS,
    )
    return run(tab_t, tail8)


@jax.jit
def _emb_lookup(x, tab5):
    run = pl.kernel(
        _emb_body,
        out_type=jax.ShapeDtypeStruct((OUT_W, BATCH), jnp.float32),
        mesh=_MESH,
        scratch_types=[
            pltpu.VMEM((BC, XW), jnp.int32),
            pltpu.VMEM((NF, BC), jnp.int32),
            pltpu.VMEM((NF, BC), jnp.int32),
            pltpu.VMEM((2 * BC, BLK), jnp.float32),
            pltpu.VMEM((OUT_W, BC), jnp.float32),
            pltpu.SemaphoreType.DMA,
            pltpu.SemaphoreType.DMA,
        ],
        compiler_params=_PARAMS,
    )
    return run(x, tab5)


def kernel(x, tables):
    tail8 = jnp.pad(
        tables[:, TAIL_V0:, :].reshape(NF, TAIL_ROWS, BLK),
        ((0, 0), (0, TAIL_PAD - TAIL_ROWS), (0, 0)),
    )
    tab_lin = _transpose_tables(tables.transpose(0, 2, 1), tail8)
    return _emb_lookup(x, tab_lin).T


# speedup vs baseline: 1.0688x; 1.0008x over previous
import jax
import jax.numpy as jnp
from jax import lax
from jax.experimental import pallas as pl
from jax.experimental.pallas import tpu as pltpu
from jax.experimental.pallas import tpu_sc as plsc

BATCH = 16384
NF = 26
D = 16
VOCAB = 100000
XW = 2 * NF
OUT_W = NF * D + NF
BLK = 128
VPB = BLK // D
NBLK = 12512
WIN = 1024
NCH = 97
N_ITEMS = NF * NCH
TAIL_V0 = NCH * WIN
TAIL_ROWS = (VOCAB - TAIL_V0) * D // BLK
TAIL_PAD = 88

NC = 2
NS = 16
NW = NC * NS
B_PER_W = BATCH // NW
BC = 128
N_CHUNK = B_PER_W // BC
L = 16


def _tr_body(tab_t_hbm, tail_hbm, out_hbm, in_v, out_v, idx_w, gsem, osem):
    wid = lax.axis_index("s") * NC + lax.axis_index("c")
    iota = lax.iota(jnp.int32, L)
    ni = (N_ITEMS - wid + NW - 1) // NW
    ROWS = WIN // VPB

    def load(item, slot):
        f = item // NCH
        k = item - f * NCH
        for g in range(2):
            pltpu.async_copy(
                tab_t_hbm.at[f, pl.ds(g * VPB, VPB), pl.ds(k * WIN, WIN)],
                in_v.at[pl.ds(slot * D + g * VPB, VPB), :],
                gsem,
            )

    def absorb_in():
        for g in range(2):
            pltpu.make_async_copy(
                tab_t_hbm.at[0, pl.ds(0, VPB), pl.ds(0, WIN)],
                in_v.at[pl.ds(g * VPB, VPB), :],
                gsem,
            ).wait()

    def absorb_out():
        pltpu.make_async_copy(
            out_v.at[pl.ds(0, ROWS), :],
            out_hbm.at[0].at[idx_w.at[0]],
            osem,
        ).wait()

    @pl.when(ni > 0)
    def _():
        load(wid, 0)

    @pl.when(ni > 1)
    def _():
        load(wid + NW, 1)

    @pl.when(ni > 2)
    def _():
        load(wid + 2 * NW, 2)

    def step(i, carry):
        item = wid + i * NW
        f = item // NCH
        k = item - f * NCH

        @pl.when(i + 3 < ni)
        def _():
            load(item + 3 * NW, (i + 3) % 4)

        absorb_in()

        rowvec = (i % 4) * D + iota
        s_out = (i % 2) * ROWS

        def row4(rr, carry2):
            for u in range(4):
                r = rr * 4 + u
                for q in range(VPB):
                    cols = jnp.broadcast_to(VPB * r + q, (L,))
                    vals = plsc.load_gather(in_v, [rowvec, cols])
                    out_v[s_out + r, pl.ds(q * L, L)] = vals
            return carry2

        lax.fori_loop(0, ROWS // 4, row4, 0)

        @pl.when(i >= 2)
        def _():
            absorb_out()

        for j0 in range(ROWS // L):
            idx_w[i % 2, pl.ds(j0 * L, L)] = k * ROWS + j0 * L + iota
        pltpu.async_copy(
            out_v.at[pl.ds(s_out, ROWS), :],
            out_hbm.at[f].at[idx_w.at[i % 2]],
            osem,
        )
        return carry

    lax.fori_loop(0, ni, step, 0)

    @pl.when(ni >= 2)
    def _():
        absorb_out()

    @pl.when(ni >= 1)
    def _():
        absorb_out()

    @pl.when(wid < NF)
    def _():
        pltpu.sync_copy(tail_hbm.at[wid], out_v.at[pl.ds(0, TAIL_PAD), :])
        pltpu.sync_copy(
            out_v.at[pl.ds(0, TAIL_PAD), :],
            out_hbm.at[wid, pl.ds(TAIL_V0 // VPB, TAIL_PAD), :],
        )


def _emb_body(x_hbm, tab_hbm, out_hbm, xv, blk_v, low_v, bbuf, stage_v, gsem, osem):
    wid = lax.axis_index("s") * NC + lax.axis_index("c")
    iota = lax.iota(jnp.int32, L)

    def chunk(c, carry):
        base = wid * B_PER_W + c * BC

        pltpu.sync_copy(x_hbm.at[pl.ds(base, BC), :], xv)

        for j in range(NF * BC // L):
            f = j // (BC // L)
            b0 = (j % (BC // L)) * L
            vals = plsc.load_gather(
                xv, [b0 + iota, jnp.full((L,), f, jnp.int32)]
            )
            blk_v[f, pl.ds(b0, L)] = vals >> 3
            low_v[f, pl.ds(b0, L)] = (vals & 7) << 4

        for j in range(NF * BC // L):
            col = j // (BC // L)
            b0 = (j % (BC // L)) * L
            vals = plsc.load_gather(
                xv, [b0 + iota, jnp.full((L,), NF + col, jnp.int32)]
            )
            stage_v[NF * D + col, pl.ds(b0, L)] = vals.astype(jnp.float32)

        pltpu.async_copy(
            tab_hbm.at[0].at[blk_v.at[0]], bbuf.at[pl.ds(0, BC), :], gsem
        )

        def feat(f, carry2):
            nxt = f + 1

            @pl.when(nxt < NF)
            def _():
                pltpu.async_copy(
                    tab_hbm.at[nxt].at[blk_v.at[nxt]],
                    bbuf.at[pl.ds((nxt % 2) * BC, BC), :],
                    gsem,
                )

            pltpu.make_async_copy(
                tab_hbm.at[0].at[blk_v.at[0]],
                bbuf.at[pl.ds((f % 2) * BC, BC), :],
                gsem,
            ).wait()

            bufbase = (f % 2) * BC
            for b0 in range(BC // L):
                rows = bufbase + b0 * L + iota
                lowvec = low_v[f, pl.ds(b0 * L, L)]
                for d in range(D):
                    vals = plsc.load_gather(bbuf, [rows, lowvec + d])
                    stage_v[f * D + d, pl.ds(b0 * L, L)] = vals
            return carry2

        lax.fori_loop(0, NF, feat, 0)

        pltpu.async_copy(stage_v, out_hbm.at[:, pl.ds(base, BC)], osem).wait()
        return carry

    lax.fori_loop(0, N_CHUNK, chunk, 0)


_MESH = plsc.VectorSubcoreMesh(
    core_axis_name="c", subcore_axis_name="s", num_cores=NC, num_subcores=NS
)
_PARAMS = pltpu.CompilerParams(
    use_tc_tiling_on_sc=True, needs_layout_passes=False
)


@jax.jit
def _transpose_tables(tab_t, tail8):
    run = pl.kernel(
        _tr_body,
        out_type=jax.ShapeDtypeStruct((NF, NBLK, BLK), jnp.float32),
        mesh=_MESH,
        scratch_types=[
            pltpu.VMEM((4 * D, WIN), jnp.float32),
            pltpu.VMEM((2 * (WIN // VPB), BLK), jnp.float32),
            pltpu.VMEM((2, WIN // VPB), jnp.int32),
            pltpu.SemaphoreType.DMA,
            pltpu.SemaphoreType.DMA,
        ],
        compiler_params=_PARAMS,
    )
    return run(tab_t, tail8)


@jax.jit
def _emb_lookup(x, tab5):
    run = pl.kernel(
        _emb_body,
        out_type=jax.ShapeDtypeStruct((OUT_W, BATCH), jnp.float32),
        mesh=_MESH,
        scratch_types=[
            pltpu.VMEM((BC, XW), jnp.int32),
            pltpu.VMEM((NF, BC), jnp.int32),
            pltpu.VMEM((NF, BC), jnp.int32),
            pltpu.VMEM((2 * BC, BLK), jnp.float32),
            pltpu.VMEM((OUT_W, BC), jnp.float32),
            pltpu.SemaphoreType.DMA,
            pltpu.SemaphoreType.DMA,
        ],
        compiler_params=_PARAMS,
    )
    return run(x, tab5)


def kernel(x, tables):
    tail8 = jnp.pad(
        tables[:, TAIL_V0:, :].reshape(NF, TAIL_ROWS, BLK),
        ((0, 0), (0, TAIL_PAD - TAIL_ROWS), (0, 0)),
    )
    tab_lin = _transpose_tables(tables.transpose(0, 2, 1), tail8)
    return _emb_lookup(x, tab_lin).T


# speedup vs baseline: 1.0968x; 1.0262x over previous
import functools

import jax
import jax.numpy as jnp
from jax import lax
from jax.experimental import pallas as pl
from jax.experimental.pallas import tpu as pltpu
from jax.experimental.pallas import tpu_sc as plsc

BATCH = 16384
NF = 26
D = 16
VOCAB = 100000
XW = 2 * NF
OUT_W = NF * D + NF

NC = 2
NS = 16
NW = NC * NS
B_PER_W = BATCH // NW
BC = 128
N_CHUNK = B_PER_W // BC
L = 16


def _body(x_hbm, tab_hbm, out_hbm, xv, idx_v, emb_v, cont_a, gsem, osem):
    wid = lax.axis_index("s") * NC + lax.axis_index("c")
    iota = lax.iota(jnp.int32, L)
    def chunk(c, carry):
        base = wid * B_PER_W + c * BC

        pltpu.sync_copy(x_hbm.at[pl.ds(base, BC), :], xv)

        for j in range(NF * BC // L):
            f = j // (BC // L)
            b0 = (j % (BC // L)) * L
            vals = plsc.load_gather(xv, [b0 + iota, jnp.full((L,), f, jnp.int32)])
            idx_v[pl.ds(j * L, L)] = vals

        gathers = [
            pltpu.async_copy(
                tab_hbm.at[f].at[idx_v.at[pl.ds(f * BC, BC)]],
                emb_v.at[pl.ds(f * BC, BC), :],
                gsem,
            )
            for f in range(NF)
        ]

        for j in range(NF * BC // L):
            p = j * L + iota
            b = p // NF
            col = p - b * NF
            vals = plsc.load_gather(xv, [b, col + NF])
            plsc.store_scatter(cont_a, [b, col], vals.astype(jnp.float32))

        for g in gathers:
            g.wait()

        outs = [
            pltpu.async_copy(
                emb_v.at[pl.ds(f * BC, BC), :],
                out_hbm.at[pl.ds(base, BC), pl.ds(f * D, D)],
                osem,
            )
            for f in range(NF)
        ]
        outs.append(
            pltpu.async_copy(
                cont_a, out_hbm.at[pl.ds(base, BC), pl.ds(NF * D, NF)], osem
            )
        )
        for o in outs:
            o.wait()
        return carry

    lax.fori_loop(0, N_CHUNK, chunk, 0)


@jax.jit
def _emb_lookup(x_2d, tab3):
    run = pl.kernel(
        _body,
        out_type=jax.ShapeDtypeStruct((BATCH, OUT_W), jnp.float32),
        mesh=plsc.VectorSubcoreMesh(
            core_axis_name="c", subcore_axis_name="s", num_cores=NC,
            num_subcores=NS,
        ),
        scratch_types=[
            pltpu.VMEM((BC, XW), jnp.int32),
            pltpu.VMEM((NF * BC,), jnp.int32),
            pltpu.VMEM((NF * BC, D), jnp.float32),
            pltpu.VMEM((BC, NF), jnp.float32),
            pltpu.SemaphoreType.DMA,
            pltpu.SemaphoreType.DMA,
        ],
        compiler_params=pltpu.CompilerParams(
            use_tc_tiling_on_sc=False, needs_layout_passes=False
        ),
    )
    return run(x_2d, tab3)


def kernel(x, tables):
    return _emb_lookup(x, tables)
